# initial kernel scaffold (unmeasured)
import jax
import jax.numpy as jnp
from jax import lax
from jax.experimental import pallas as pl
from jax.experimental.pallas import tpu as pltpu

N_DEV = 32
STEPS = 5


def kernel(x, Wq, Wo, K_ext, V_ext):
    B, Sq, D = x.shape
    _, Skv, Hkv, Dh = K_ext.shape
    Hq = D // Dh // (D // Dh // 8) if False else 8
    GROUP = Hq // Hkv
    BH = B * Hq

    xb = x.astype(jnp.bfloat16)
    wqb = Wq.astype(jnp.bfloat16)
    wob = Wo.astype(jnp.bfloat16)
    kb = jnp.transpose(K_ext, (0, 2, 1, 3)).astype(jnp.bfloat16)
    vb = jnp.transpose(V_ext, (0, 2, 1, 3)).astype(jnp.bfloat16)

    def body(x_ref, wq_ref, wo_ref, k_ref, v_ref, out_ref,
             acc_ref, ml_ref, accs_ref, accr_ref, mls_ref, mlr_ref,
             a_send_sems, a_recv_sems, m_send_sems, m_recv_sems):
        my = lax.axis_index("i")

        barrier = pltpu.get_barrier_semaphore()
        for s in range(STEPS):
            pl.semaphore_signal(
                barrier, inc=1,
                device_id=(my ^ (1 << s),),
                device_id_type=pl.DeviceIdType.MESH,
            )
        pl.semaphore_wait(barrier, STEPS)

        ms, ls = [], []
        for b in range(B):
            q_all = lax.dot_general(
                x_ref[b], wq_ref[...], (((1,), (0,)), ((), ())),
                preferred_element_type=jnp.float32,
            )
            q_all = (q_all * 0.125).astype(jnp.bfloat16)
            for h in range(Hq):
                bh = b * Hq + h
                q = q_all[:, h * Dh:(h + 1) * Dh]
                k = k_ref[b, h // GROUP]
                v = v_ref[b, h // GROUP]
                s_t = lax.dot_general(
                    k, q, (((1,), (1,)), ((), ())),
                    preferred_element_type=jnp.float32,
                )
                m = jnp.max(s_t, axis=0, keepdims=True)
                p = jnp.exp(s_t - m)
                l = jnp.sum(p, axis=0, keepdims=True)
                acc_ref[bh] = lax.dot_general(
                    v, p.astype(jnp.bfloat16), (((0,), (0,)), ((), ())),
                    preferred_element_type=jnp.float32,
                )
                ms.append(m)
                ls.append(l)
        ml_ref[0] = jnp.concatenate(ms, axis=0)
        ml_ref[1] = jnp.concatenate(ls, axis=0)

        for s in range(STEPS):
            partner = my ^ (1 << s)
            accs_ref[...] = acc_ref[...].astype(jnp.bfloat16)
            mls_ref[...] = ml_ref[...]
            rdma_a = pltpu.make_async_remote_copy(
                src_ref=accs_ref, dst_ref=accr_ref.at[s],
                send_sem=a_send_sems.at[s], recv_sem=a_recv_sems.at[s],
                device_id=(partner,), device_id_type=pl.DeviceIdType.MESH,
            )
            rdma_m = pltpu.make_async_remote_copy(
                src_ref=mls_ref, dst_ref=mlr_ref.at[s],
                send_sem=m_send_sems.at[s], recv_sem=m_recv_sems.at[s],
                device_id=(partner,), device_id_type=pl.DeviceIdType.MESH,
            )
            rdma_a.start()
            rdma_m.start()
            rdma_a.wait()
            rdma_m.wait()

            m_cur, l_cur = ml_ref[0], ml_ref[1]
            m_rcv, l_rcv = mlr_ref[s, 0], mlr_ref[s, 1]
            m_new = jnp.maximum(m_cur, m_rcv)
            a1 = jnp.exp(m_cur - m_new)
            a2 = jnp.exp(m_rcv - m_new)
            ml_ref[0] = m_new
            ml_ref[1] = l_cur * a1 + l_rcv * a2
            acc_ref[...] = (
                acc_ref[...] * a1[:, None, :]
                + accr_ref[s].astype(jnp.float32) * a2[:, None, :]
            )

        linv = 1.0 / ml_ref[1]
        for b in range(B):
            rows = []
            for h in range(Hq):
                bh = b * Hq + h
                rows.append(
                    (acc_ref[bh] * linv[bh:bh + 1, :]).astype(jnp.bfloat16)
                )
            att_t = jnp.concatenate(rows, axis=0)
            out_ref[b] = lax.dot_general(
                att_t, wo_ref[...], (((0,), (0,)), ((), ())),
                preferred_element_type=jnp.float32,
            )

    return pl.pallas_call(
        body,
        out_shape=jax.ShapeDtypeStruct((B, Sq, D), jnp.float32),
        in_specs=[pl.BlockSpec(memory_space=pltpu.VMEM)] * 5,
        out_specs=pl.BlockSpec(memory_space=pltpu.VMEM),
        scratch_shapes=[
            pltpu.VMEM((BH, Dh, Sq), jnp.float32),
            pltpu.VMEM((2, BH, Sq), jnp.float32),
            pltpu.VMEM((BH, Dh, Sq), jnp.bfloat16),
            pltpu.VMEM((STEPS, BH, Dh, Sq), jnp.bfloat16),
            pltpu.VMEM((2, BH, Sq), jnp.float32),
            pltpu.VMEM((STEPS, 2, BH, Sq), jnp.float32),
            pltpu.SemaphoreType.DMA((STEPS,)),
            pltpu.SemaphoreType.DMA((STEPS,)),
            pltpu.SemaphoreType.DMA((STEPS,)),
            pltpu.SemaphoreType.DMA((STEPS,)),
        ],
        compiler_params=pltpu.CompilerParams(collective_id=0),
    )(xb, wqb, wob, kb, vb)


# baseline (device time: 45473 ns/iter reference)
import jax
import jax.numpy as jnp
from jax import lax
from jax.experimental import pallas as pl
from jax.experimental.pallas import tpu as pltpu

N_DEV = 32
STEPS = 5


def kernel(x, Wq, Wo, K_ext, V_ext):
    B, Sq, D = x.shape
    _, Skv, Hkv, Dh = K_ext.shape
    Hq = D // Dh
    GROUP = Hq // Hkv
    BH = B * Hq

    xb = x.astype(jnp.bfloat16)
    wqb = Wq.astype(jnp.bfloat16)
    wob = Wo.astype(jnp.bfloat16)
    kb = jnp.transpose(K_ext, (0, 2, 1, 3)).astype(jnp.bfloat16)
    vb = jnp.transpose(V_ext, (0, 2, 1, 3)).astype(jnp.bfloat16)

    def body(x_ref, wq_ref, wo_ref, k_ref, v_ref, out_ref,
             acc_ref, ml_ref, accs_ref, accr_ref, mls_ref, mlr_ref,
             a_send_sems, a_recv_sems, m_send_sems, m_recv_sems):
        my = lax.axis_index("i")

        barrier = pltpu.get_barrier_semaphore()
        for s in range(STEPS):
            pl.semaphore_signal(
                barrier, inc=1,
                device_id=(my ^ (1 << s),),
                device_id_type=pl.DeviceIdType.MESH,
            )
        pl.semaphore_wait(barrier, STEPS)

        ms, ls = [], []
        for b in range(B):
            q_all = lax.dot_general(
                x_ref[b], wq_ref[...], (((1,), (0,)), ((), ())),
                preferred_element_type=jnp.float32,
            )
            q_all = (q_all * 0.125).astype(jnp.bfloat16)
            for h in range(Hq):
                bh = b * Hq + h
                q = q_all[:, h * Dh:(h + 1) * Dh]
                k = k_ref[b, h // GROUP]
                v = v_ref[b, h // GROUP]
                s_t = lax.dot_general(
                    k, q, (((1,), (1,)), ((), ())),
                    preferred_element_type=jnp.float32,
                )
                m = jnp.max(s_t, axis=0, keepdims=True)
                p = jnp.exp(s_t - m)
                l = jnp.sum(p, axis=0, keepdims=True)
                acc_ref[bh] = lax.dot_general(
                    v, p.astype(jnp.bfloat16), (((0,), (0,)), ((), ())),
                    preferred_element_type=jnp.float32,
                )
                ms.append(m)
                ls.append(l)
        ml_ref[0] = jnp.concatenate(ms, axis=0)
        ml_ref[1] = jnp.concatenate(ls, axis=0)

        for s in range(STEPS):
            partner = my ^ (1 << s)
            accs_ref[...] = acc_ref[...].astype(jnp.bfloat16)
            mls_ref[...] = ml_ref[...]
            rdma_a = pltpu.make_async_remote_copy(
                src_ref=accs_ref, dst_ref=accr_ref.at[s],
                send_sem=a_send_sems.at[s], recv_sem=a_recv_sems.at[s],
                device_id=(partner,), device_id_type=pl.DeviceIdType.MESH,
            )
            rdma_m = pltpu.make_async_remote_copy(
                src_ref=mls_ref, dst_ref=mlr_ref.at[s],
                send_sem=m_send_sems.at[s], recv_sem=m_recv_sems.at[s],
                device_id=(partner,), device_id_type=pl.DeviceIdType.MESH,
            )
            rdma_a.start()
            rdma_m.start()
            rdma_a.wait()
            rdma_m.wait()

            m_cur, l_cur = ml_ref[0], ml_ref[1]
            m_rcv, l_rcv = mlr_ref[s, 0], mlr_ref[s, 1]
            m_new = jnp.maximum(m_cur, m_rcv)
            a1 = jnp.exp(m_cur - m_new)
            a2 = jnp.exp(m_rcv - m_new)
            ml_ref[0] = m_new
            ml_ref[1] = l_cur * a1 + l_rcv * a2
            acc_ref[...] = (
                acc_ref[...] * a1[:, None, :]
                + accr_ref[s].astype(jnp.float32) * a2[:, None, :]
            )

        linv = 1.0 / ml_ref[1]
        for b in range(B):
            rows = []
            for h in range(Hq):
                bh = b * Hq + h
                rows.append(
                    (acc_ref[bh] * linv[bh:bh + 1, :]).astype(jnp.bfloat16)
                )
            att_t = jnp.concatenate(rows, axis=0)
            out_ref[b] = lax.dot_general(
                att_t, wo_ref[...], (((0,), (0,)), ((), ())),
                preferred_element_type=jnp.float32,
            )

    return pl.pallas_call(
        body,
        out_shape=jax.ShapeDtypeStruct((B, Sq, D), jnp.float32),
        in_specs=[pl.BlockSpec(memory_space=pltpu.VMEM)] * 5,
        out_specs=pl.BlockSpec(memory_space=pltpu.VMEM),
        scratch_shapes=[
            pltpu.VMEM((BH, Dh, Sq), jnp.float32),
            pltpu.VMEM((2, BH, Sq), jnp.float32),
            pltpu.VMEM((BH, Dh, Sq), jnp.bfloat16),
            pltpu.VMEM((STEPS, BH, Dh, Sq), jnp.bfloat16),
            pltpu.VMEM((2, BH, Sq), jnp.float32),
            pltpu.VMEM((STEPS, 2, BH, Sq), jnp.float32),
            pltpu.SemaphoreType.DMA((STEPS,)),
            pltpu.SemaphoreType.DMA((STEPS,)),
            pltpu.SemaphoreType.DMA((STEPS,)),
            pltpu.SemaphoreType.DMA((STEPS,)),
        ],
        compiler_params=pltpu.CompilerParams(collective_id=0),
    )(xb, wqb, wob, kb, vb)


# device time: 44380 ns/iter; 1.0246x vs baseline; 1.0246x over previous
import jax
import jax.numpy as jnp
from jax import lax
from jax.experimental import pallas as pl
from jax.experimental.pallas import tpu as pltpu

N_DEV = 32
STEPS = 5


def kernel(x, Wq, Wo, K_ext, V_ext):
    B, Sq, D = x.shape
    _, Skv, Hkv, Dh = K_ext.shape
    Hq = D // Dh
    GROUP = Hq // Hkv
    BH = B * Hq

    xb = x.astype(jnp.bfloat16)
    wqb = Wq.astype(jnp.bfloat16)
    wob = Wo.astype(jnp.bfloat16)
    kb = jnp.transpose(K_ext, (0, 2, 1, 3)).astype(jnp.bfloat16)
    vb = jnp.transpose(V_ext, (0, 2, 1, 3)).astype(jnp.bfloat16)

    def body(x_ref, wq_ref, wo_ref, k_ref, v_ref, out_ref,
             acc_ref, l_ref, accr_ref, lr_ref,
             a_send_sems, a_recv_sems, l_send_sems, l_recv_sems):
        my = lax.axis_index("i")

        barrier = pltpu.get_barrier_semaphore()
        for s in range(STEPS):
            pl.semaphore_signal(
                barrier, inc=1,
                device_id=(my ^ (1 << s),),
                device_id_type=pl.DeviceIdType.MESH,
            )
        pl.semaphore_wait(barrier, STEPS)

        ls = []
        for b in range(B):
            q_all = lax.dot_general(
                x_ref[b], wq_ref[...], (((1,), (0,)), ((), ())),
                preferred_element_type=jnp.float32,
            )
            q_all = (q_all * 0.125).astype(jnp.bfloat16)
            for h in range(Hq):
                bh = b * Hq + h
                q = q_all[:, h * Dh:(h + 1) * Dh]
                k = k_ref[b, h // GROUP]
                v = v_ref[b, h // GROUP]
                s_t = lax.dot_general(
                    k, q, (((1,), (1,)), ((), ())),
                    preferred_element_type=jnp.float32,
                )
                p = jnp.exp(s_t)
                ls.append(jnp.sum(p, axis=0, keepdims=True))
                acc_ref[0, bh] = lax.dot_general(
                    v, p.astype(jnp.bfloat16), (((0,), (0,)), ((), ())),
                    preferred_element_type=jnp.float32,
                ).astype(jnp.bfloat16)
        l_ref[0] = jnp.concatenate(ls, axis=0)

        prev = []
        for s in range(STEPS):
            cur, nxt = s % 2, (s + 1) % 2
            partner = my ^ (1 << s)
            rdma_a = pltpu.make_async_remote_copy(
                src_ref=acc_ref.at[cur], dst_ref=accr_ref.at[s],
                send_sem=a_send_sems.at[s], recv_sem=a_recv_sems.at[s],
                device_id=(partner,), device_id_type=pl.DeviceIdType.MESH,
            )
            rdma_l = pltpu.make_async_remote_copy(
                src_ref=l_ref.at[cur], dst_ref=lr_ref.at[s],
                send_sem=l_send_sems.at[s], recv_sem=l_recv_sems.at[s],
                device_id=(partner,), device_id_type=pl.DeviceIdType.MESH,
            )
            rdma_a.start()
            rdma_l.start()
            for r in prev:
                r.wait_send()
            prev = [rdma_a, rdma_l]
            rdma_a.wait_recv()
            rdma_l.wait_recv()
            acc_ref[nxt] = (
                acc_ref[cur].astype(jnp.float32)
                + accr_ref[s].astype(jnp.float32)
            ).astype(jnp.bfloat16)
            l_ref[nxt] = l_ref[cur] + lr_ref[s]
        for r in prev:
            r.wait_send()

        fin = STEPS % 2
        linv = 1.0 / l_ref[fin]
        for b in range(B):
            rows = []
            for h in range(Hq):
                bh = b * Hq + h
                rows.append(
                    (acc_ref[fin, bh].astype(jnp.float32)
                     * linv[bh:bh + 1, :]).astype(jnp.bfloat16)
                )
            att_t = jnp.concatenate(rows, axis=0)
            out_ref[b] = lax.dot_general(
                att_t, wo_ref[...], (((0,), (0,)), ((), ())),
                preferred_element_type=jnp.float32,
            )

    return pl.pallas_call(
        body,
        out_shape=jax.ShapeDtypeStruct((B, Sq, D), jnp.float32),
        in_specs=[pl.BlockSpec(memory_space=pltpu.VMEM)] * 5,
        out_specs=pl.BlockSpec(memory_space=pltpu.VMEM),
        scratch_shapes=[
            pltpu.VMEM((2, BH, Dh, Sq), jnp.bfloat16),
            pltpu.VMEM((2, BH, Sq), jnp.float32),
            pltpu.VMEM((STEPS, BH, Dh, Sq), jnp.bfloat16),
            pltpu.VMEM((STEPS, BH, Sq), jnp.float32),
            pltpu.SemaphoreType.DMA((STEPS,)),
            pltpu.SemaphoreType.DMA((STEPS,)),
            pltpu.SemaphoreType.DMA((STEPS,)),
            pltpu.SemaphoreType.DMA((STEPS,)),
        ],
        compiler_params=pltpu.CompilerParams(collective_id=0),
    )(xb, wqb, wob, kb, vb)


# device time: 34411 ns/iter; 1.3215x vs baseline; 1.2897x over previous
import jax
import jax.numpy as jnp
from jax import lax
from jax.experimental import pallas as pl
from jax.experimental.pallas import tpu as pltpu

N_DEV = 32
STEPS = 5
CHAINS = 2
DIMS = [[0, 1, 2, 3, 4],
        [3, 4, 0, 1, 2]]


def kernel(x, Wq, Wo, K_ext, V_ext):
    B, Sq, D = x.shape
    _, Skv, Hkv, Dh = K_ext.shape
    Hq = D // Dh
    GROUP = Hq // Hkv
    LROW = Hq * Dh
    ROWS = Hq * Dh + 16

    xb = x.astype(jnp.bfloat16)
    wqb = Wq.astype(jnp.bfloat16)
    wob = Wo.astype(jnp.bfloat16)
    kb = jnp.transpose(K_ext, (0, 2, 1, 3)).astype(jnp.bfloat16)
    vb = jnp.transpose(V_ext, (0, 2, 1, 3)).astype(jnp.bfloat16)

    def body(x_ref, wq_ref, wo_ref, k_ref, v_ref, out_ref,
             st_ref, recv_ref, send_sems, recv_sems):
        my = lax.axis_index("i")

        barrier = pltpu.get_barrier_semaphore()
        for d in range(STEPS):
            pl.semaphore_signal(
                barrier, inc=1,
                device_id=(my ^ (1 << d),),
                device_id_type=pl.DeviceIdType.MESH,
            )
        pl.semaphore_wait(barrier, STEPS)

        def make_rdma(c, i):
            partner = my ^ (1 << DIMS[c][i])
            return pltpu.make_async_remote_copy(
                src_ref=st_ref.at[c, i % 2], dst_ref=recv_ref.at[c, i],
                send_sem=send_sems.at[c * STEPS + i],
                recv_sem=recv_sems.at[c * STEPS + i],
                device_id=(partner,), device_id_type=pl.DeviceIdType.MESH,
            )

        rdmas = {}
        for b in range(B):
            q_all = lax.dot_general(
                x_ref[b], wq_ref[...], (((1,), (0,)), ((), ())),
                preferred_element_type=jnp.float32,
            )
            q_all = (q_all * 0.125).astype(jnp.bfloat16)
            ls = []
            for h in range(Hq):
                q = q_all[:, h * Dh:(h + 1) * Dh]
                k = k_ref[b, h // GROUP]
                v = v_ref[b, h // GROUP]
                s_t = lax.dot_general(
                    k, q, (((1,), (1,)), ((), ())),
                    preferred_element_type=jnp.float32,
                )
                p = jnp.exp(s_t)
                ls.append(jnp.sum(p, axis=0, keepdims=True))
                st_ref[b, 0, h * Dh:(h + 1) * Dh, :] = lax.dot_general(
                    v, p.astype(jnp.bfloat16), (((0,), (0,)), ((), ())),
                    preferred_element_type=jnp.float32,
                ).astype(jnp.bfloat16)
            lpack = jnp.concatenate(ls + ls, axis=0).astype(jnp.bfloat16)
            st_ref[b, 0, LROW:ROWS, :] = lpack
            r = make_rdma(b, 0)
            r.start()
            rdmas[(b, 0)] = r

        for i in range(STEPS):
            for c in range(CHAINS):
                rdmas[(c, i)].wait_recv()
                if i >= 1:
                    rdmas[(c, i - 1)].wait_send()
                st_ref[c, (i + 1) % 2] = (
                    st_ref[c, i % 2].astype(jnp.float32)
                    + recv_ref[c, i].astype(jnp.float32)
                ).astype(jnp.bfloat16)
                if i + 1 < STEPS:
                    r = make_rdma(c, i + 1)
                    r.start()
                    rdmas[(c, i + 1)] = r
        for c in range(CHAINS):
            rdmas[(c, STEPS - 1)].wait_send()

        fin = STEPS % 2
        for b in range(B):
            st = st_ref[b, fin]
            linv = 1.0 / st[LROW:LROW + Hq, :].astype(jnp.float32)
            rows = []
            for h in range(Hq):
                rows.append(
                    (st[h * Dh:(h + 1) * Dh, :].astype(jnp.float32)
                     * linv[h:h + 1, :]).astype(jnp.bfloat16))
            att_t = jnp.concatenate(rows, axis=0)
            out_ref[b] = lax.dot_general(
                att_t, wo_ref[...], (((0,), (0,)), ((), ())),
                preferred_element_type=jnp.float32,
            )

    return pl.pallas_call(
        body,
        out_shape=jax.ShapeDtypeStruct((B, Sq, D), jnp.float32),
        in_specs=[pl.BlockSpec(memory_space=pltpu.VMEM)] * 5,
        out_specs=pl.BlockSpec(memory_space=pltpu.VMEM),
        scratch_shapes=[
            pltpu.VMEM((CHAINS, 2, ROWS, Sq), jnp.bfloat16),
            pltpu.VMEM((CHAINS, STEPS, ROWS, Sq), jnp.bfloat16),
            pltpu.SemaphoreType.DMA((CHAINS * STEPS,)),
            pltpu.SemaphoreType.DMA((CHAINS * STEPS,)),
        ],
        compiler_params=pltpu.CompilerParams(collective_id=0),
    )(xb, wqb, wob, kb, vb)


# device time: 29260 ns/iter; 1.5541x vs baseline; 1.1760x over previous
import jax
import jax.numpy as jnp
from jax import lax
from jax.experimental import pallas as pl
from jax.experimental.pallas import tpu as pltpu

N_DEV = 32
STEPS = 5
ROT = [0, 3, 1, 4]
NS = len(ROT)


def kernel(x, Wq, Wo, K_ext, V_ext):
    B, Sq, D = x.shape
    _, Skv, Hkv, Dh = K_ext.shape
    Hq = D // Dh
    GROUP = Hq // Hkv
    HG = Hq // 2
    LROW = HG * Dh
    ROWS = LROW + 16

    xb = x.astype(jnp.bfloat16)
    wqb = Wq.astype(jnp.bfloat16)
    wob = Wo.astype(jnp.bfloat16)
    kb = jnp.transpose(K_ext, (0, 2, 1, 3)).astype(jnp.bfloat16)
    vb = jnp.transpose(V_ext, (0, 2, 1, 3)).astype(jnp.bfloat16)

    def body(x_ref, wq_ref, wo_ref, k_ref, v_ref, out_ref,
             st_ref, recv_ref, send_sems, recv_sems):
        my = lax.axis_index("i")

        barrier = pltpu.get_barrier_semaphore()
        for d in range(STEPS):
            pl.semaphore_signal(
                barrier, inc=1,
                device_id=(my ^ (1 << d),),
                device_id_type=pl.DeviceIdType.MESH,
            )
        pl.semaphore_wait(barrier, STEPS)

        def make_rdma(k, i):
            partner = my ^ (1 << ((i + ROT[k]) % STEPS))
            return pltpu.make_async_remote_copy(
                src_ref=st_ref.at[k, i % 2], dst_ref=recv_ref.at[k, i],
                send_sem=send_sems.at[k * STEPS + i],
                recv_sem=recv_sems.at[k * STEPS + i],
                device_id=(partner,), device_id_type=pl.DeviceIdType.MESH,
            )

        rdmas = {}
        for b in range(B):
            q_all = lax.dot_general(
                x_ref[b], wq_ref[...], (((1,), (0,)), ((), ())),
                preferred_element_type=jnp.float32,
            )
            q_all = (q_all * 0.125).astype(jnp.bfloat16)
            for g in range(2):
                k_stream = b * 2 + g
                ls = []
                for j in range(HG):
                    h = g * HG + j
                    q = q_all[:, h * Dh:(h + 1) * Dh]
                    kk = k_ref[b, h // GROUP]
                    vv = v_ref[b, h // GROUP]
                    s_t = lax.dot_general(
                        kk, q, (((1,), (1,)), ((), ())),
                        preferred_element_type=jnp.float32,
                    )
                    p = jnp.exp(s_t)
                    ls.append(jnp.sum(p, axis=0, keepdims=True))
                    st_ref[k_stream, 0, j * Dh:(j + 1) * Dh, :] = (
                        lax.dot_general(
                            vv, p.astype(jnp.bfloat16),
                            (((0,), (0,)), ((), ())),
                            preferred_element_type=jnp.float32,
                        ).astype(jnp.bfloat16))
                lpack = jnp.concatenate(ls * 4, axis=0).astype(jnp.bfloat16)
                st_ref[k_stream, 0, LROW:ROWS, :] = lpack
                r = make_rdma(k_stream, 0)
                r.start()
                rdmas[(k_stream, 0)] = r

        for i in range(STEPS):
            for k in range(NS):
                rdmas[(k, i)].wait_recv()
                if i >= 1:
                    rdmas[(k, i - 1)].wait_send()
                st_ref[k, (i + 1) % 2] = (
                    st_ref[k, i % 2].astype(jnp.float32)
                    + recv_ref[k, i].astype(jnp.float32)
                ).astype(jnp.bfloat16)
                if i + 1 < STEPS:
                    r = make_rdma(k, i + 1)
                    r.start()
                    rdmas[(k, i + 1)] = r
        for k in range(NS):
            rdmas[(k, STEPS - 1)].wait_send()

        fin = STEPS % 2
        for b in range(B):
            rows = []
            for g in range(2):
                st = st_ref[b * 2 + g, fin]
                linv = 1.0 / st[LROW:LROW + HG, :].astype(jnp.float32)
                for j in range(HG):
                    rows.append(
                        (st[j * Dh:(j + 1) * Dh, :].astype(jnp.float32)
                         * linv[j:j + 1, :]).astype(jnp.bfloat16))
            att_t = jnp.concatenate(rows, axis=0)
            out_ref[b] = lax.dot_general(
                att_t, wo_ref[...], (((0,), (0,)), ((), ())),
                preferred_element_type=jnp.float32,
            )

    return pl.pallas_call(
        body,
        out_shape=jax.ShapeDtypeStruct((B, Sq, D), jnp.float32),
        in_specs=[pl.BlockSpec(memory_space=pltpu.VMEM)] * 5,
        out_specs=pl.BlockSpec(memory_space=pltpu.VMEM),
        scratch_shapes=[
            pltpu.VMEM((NS, 2, ROWS, Sq), jnp.bfloat16),
            pltpu.VMEM((NS, STEPS, ROWS, Sq), jnp.bfloat16),
            pltpu.SemaphoreType.DMA((NS * STEPS,)),
            pltpu.SemaphoreType.DMA((NS * STEPS,)),
        ],
        compiler_params=pltpu.CompilerParams(collective_id=0),
    )(xb, wqb, wob, kb, vb)


# device time: 27471 ns/iter; 1.6553x vs baseline; 1.0651x over previous
import jax
import jax.numpy as jnp
from jax import lax
from jax.experimental import pallas as pl
from jax.experimental.pallas import tpu as pltpu

N_DEV = 32
STEPS = 5
ROT = [0, 3, 1, 4, 2, 0, 3, 1]
NS = len(ROT)
SPLIT = NS // 2


def kernel(x, Wq, Wo, K_ext, V_ext):
    B, Sq, D = x.shape
    _, Skv, Hkv, Dh = K_ext.shape
    Hq = D // Dh
    GROUP = Hq // Hkv
    HG = Hq // SPLIT
    LROW = HG * Dh
    ROWS = LROW + 16

    xb = x.astype(jnp.bfloat16)
    wqb = Wq.astype(jnp.bfloat16)
    wob = Wo.astype(jnp.bfloat16)
    kb = jnp.transpose(K_ext, (0, 2, 1, 3)).astype(jnp.bfloat16)
    vb = jnp.transpose(V_ext, (0, 2, 1, 3)).astype(jnp.bfloat16)

    def body(x_ref, wq_ref, wo_ref, k_ref, v_ref, out_ref,
             st_ref, recv_ref, send_sems, recv_sems):
        my = lax.axis_index("i")

        barrier = pltpu.get_barrier_semaphore()
        for d in range(STEPS):
            pl.semaphore_signal(
                barrier, inc=1,
                device_id=(my ^ (1 << d),),
                device_id_type=pl.DeviceIdType.MESH,
            )
        pl.semaphore_wait(barrier, STEPS)

        def make_rdma(k, i):
            partner = my ^ (1 << ((i + ROT[k]) % STEPS))
            return pltpu.make_async_remote_copy(
                src_ref=st_ref.at[k, i % 2], dst_ref=recv_ref.at[k, i],
                send_sem=send_sems.at[k * STEPS + i],
                recv_sem=recv_sems.at[k * STEPS + i],
                device_id=(partner,), device_id_type=pl.DeviceIdType.MESH,
            )

        rdmas = {}
        for b in range(B):
            q_all = lax.dot_general(
                x_ref[b], wq_ref[...], (((1,), (0,)), ((), ())),
                preferred_element_type=jnp.float32,
            )
            q_all = (q_all * 0.125).astype(jnp.bfloat16)
            for g in range(SPLIT):
                k_stream = b * SPLIT + g
                ls = []
                for j in range(HG):
                    h = g * HG + j
                    q = q_all[:, h * Dh:(h + 1) * Dh]
                    kk = k_ref[b, h // GROUP]
                    vv = v_ref[b, h // GROUP]
                    s_t = lax.dot_general(
                        kk, q, (((1,), (1,)), ((), ())),
                        preferred_element_type=jnp.float32,
                    )
                    p = jnp.exp(s_t)
                    ls.append(jnp.sum(p, axis=0, keepdims=True))
                    st_ref[k_stream, 0, j * Dh:(j + 1) * Dh, :] = (
                        lax.dot_general(
                            vv, p.astype(jnp.bfloat16),
                            (((0,), (0,)), ((), ())),
                            preferred_element_type=jnp.float32,
                        ).astype(jnp.bfloat16))
                lpack = jnp.concatenate(
                    ls * (16 // HG), axis=0).astype(jnp.bfloat16)
                st_ref[k_stream, 0, LROW:ROWS, :] = lpack
                r = make_rdma(k_stream, 0)
                r.start()
                rdmas[(k_stream, 0)] = r

        for i in range(STEPS):
            for k in range(NS):
                rdmas[(k, i)].wait_recv()
                if i >= 1:
                    rdmas[(k, i - 1)].wait_send()
                st_ref[k, (i + 1) % 2] = (
                    st_ref[k, i % 2].astype(jnp.float32)
                    + recv_ref[k, i].astype(jnp.float32)
                ).astype(jnp.bfloat16)
                if i + 1 < STEPS:
                    r = make_rdma(k, i + 1)
                    r.start()
                    rdmas[(k, i + 1)] = r
        for k in range(NS):
            rdmas[(k, STEPS - 1)].wait_send()

        fin = STEPS % 2
        for b in range(B):
            rows = []
            for g in range(SPLIT):
                st = st_ref[b * SPLIT + g, fin]
                linv = 1.0 / st[LROW:LROW + HG, :].astype(jnp.float32)
                for j in range(HG):
                    rows.append(
                        (st[j * Dh:(j + 1) * Dh, :].astype(jnp.float32)
                         * linv[j:j + 1, :]).astype(jnp.bfloat16))
            att_t = jnp.concatenate(rows, axis=0)
            out_ref[b] = lax.dot_general(
                att_t, wo_ref[...], (((0,), (0,)), ((), ())),
                preferred_element_type=jnp.float32,
            )

    return pl.pallas_call(
        body,
        out_shape=jax.ShapeDtypeStruct((B, Sq, D), jnp.float32),
        in_specs=[pl.BlockSpec(memory_space=pltpu.VMEM)] * 5,
        out_specs=pl.BlockSpec(memory_space=pltpu.VMEM),
        scratch_shapes=[
            pltpu.VMEM((NS, 2, ROWS, Sq), jnp.bfloat16),
            pltpu.VMEM((NS, STEPS, ROWS, Sq), jnp.bfloat16),
            pltpu.SemaphoreType.DMA((NS * STEPS,)),
            pltpu.SemaphoreType.DMA((NS * STEPS,)),
        ],
        compiler_params=pltpu.CompilerParams(collective_id=0),
    )(xb, wqb, wob, kb, vb)
